# baseline (speedup 1.0000x reference)
import jax
import jax.numpy as jnp
from jax.experimental import pallas as pl

B = 8
N = 14
D = 512
CB = 512
TAU = 0.5 / (N * N)
BETA = 4.0
COEF = TAU * BETA / N


def _mask_body(x_ref, o_ref):
    m = x_ref[0, 0]
    mi = jnp.zeros((B, CB), jnp.int32)
    for i in range(N):
        for j in range(N):
            if i == 0 and j == 0:
                continue
            v = x_ref[i, j]
            pred = v > m
            m = jnp.where(pred, v, m)
            mi = jnp.where(pred, jnp.full((B, CB), i * N + j, jnp.int32), mi)

    i_max = (mi // N).astype(jnp.float32)
    j_max = (mi % N).astype(jnp.float32)

    ui = [TAU - COEF * jnp.abs(float(i) - i_max) for i in range(N)]
    wj = [COEF * jnp.abs(float(j) - j_max) for j in range(N)]

    for i in range(N):
        for j in range(N):
            mask = jnp.maximum(ui[i] - wj[j], -TAU)
            o_ref[i, j] = x_ref[i, j] * mask


@jax.jit
def _mask_layer(inputs):
    xt = inputs.transpose(1, 2, 0, 3)
    out = pl.pallas_call(
        _mask_body,
        grid=(D // CB,),
        in_specs=[pl.BlockSpec((N, N, B, CB), lambda k: (0, 0, 0, k))],
        out_specs=pl.BlockSpec((N, N, B, CB), lambda k: (0, 0, 0, k)),
        out_shape=jax.ShapeDtypeStruct((N, N, B, D), jnp.float32),
    )(xt)
    return out.transpose(2, 0, 1, 3)


def kernel(inputs):
    return _mask_layer(inputs)


# speedup vs baseline: 1.1182x; 1.1182x over previous
import jax
import jax.numpy as jnp
from jax.experimental import pallas as pl
from jax.experimental.pallas import tpu as pltpu

B = 8
N = 14
D = 512
TAU = 0.5 / (N * N)
BETA = 4.0
COEF = TAU * BETA / N


def _mask_body(x_hbm, o_hbm, x_v, y_v, ui_v, wj_v, in_sem, out_sem):
    in_cps = [
        pltpu.make_async_copy(x_hbm.at[i], x_v.at[i], in_sem.at[i])
        for i in range(N)
    ]
    for cp in in_cps:
        cp.start()

    m = None
    mi = None
    for i in range(N):
        in_cps[i].wait()
        for j in range(N):
            v = x_v[i, j]
            if m is None:
                m = v
                mi = jnp.zeros((B, D), jnp.int32)
                continue
            pred = v > m
            m = jnp.where(pred, v, m)
            mi = jnp.where(pred, jnp.full((B, D), i * N + j, jnp.int32), mi)

    i_max = (mi // N).astype(jnp.float32)
    j_max = (mi % N).astype(jnp.float32)

    for k in range(N):
        ui_v[k] = TAU - COEF * jnp.abs(float(k) - i_max)
        wj_v[k] = COEF * jnp.abs(float(k) - j_max)

    out_cps = []
    for i in range(N):
        ui = ui_v[i]
        for j in range(N):
            y_v[i, j] = x_v[i, j] * jnp.maximum(ui - wj_v[j], -TAU)
        cp = pltpu.make_async_copy(y_v.at[i], o_hbm.at[i], out_sem.at[i])
        cp.start()
        out_cps.append(cp)
    for cp in out_cps:
        cp.wait()


@jax.jit
def _mask_layer(inputs):
    xt = inputs.transpose(1, 2, 0, 3)
    out = pl.pallas_call(
        _mask_body,
        in_specs=[pl.BlockSpec(memory_space=pl.ANY)],
        out_specs=pl.BlockSpec(memory_space=pl.ANY),
        out_shape=jax.ShapeDtypeStruct((N, N, B, D), jnp.float32),
        scratch_shapes=[
            pltpu.VMEM((N, N, B, D), jnp.float32),
            pltpu.VMEM((N, N, B, D), jnp.float32),
            pltpu.VMEM((N, B, D), jnp.float32),
            pltpu.VMEM((N, B, D), jnp.float32),
            pltpu.SemaphoreType.DMA((N,)),
            pltpu.SemaphoreType.DMA((N,)),
        ],
    )(xt)
    return out.transpose(2, 0, 1, 3)


def kernel(inputs):
    return _mask_layer(inputs)
